# baseline (speedup 1.0000x reference)
import functools

import jax
import jax.numpy as jnp
from jax import lax
from jax.experimental import pallas as pl
from jax.experimental.pallas import tpu as pltpu

V = 32000; D = 1024; L = 2; H = 16; KVH = 4; HD = 64
E = 8; F = 512; FS = 512; B = 1; S = 2048
THETA = 1000000.0; EPS = 1e-6
BT = 256
NT = S // BT
BV = 1280
NV = V // BV
BTM = 512
NTM = S // BTM

f32 = jnp.float32
bf16 = jnp.bfloat16


def _bf(a):
    return a.astype(bf16)


def _mm(a, b):
    return jnp.dot(_bf(a), _bf(b), preferred_element_type=f32)


def _rms(x, g):
    return x * lax.rsqrt(jnp.mean(x * x, axis=-1, keepdims=True) + EPS) * g


def _rope(x, pos):
    half = HD // 2
    inv = 1.0 / (THETA ** (jnp.arange(0, half, dtype=jnp.float32) / half))
    ang = pos[:, None] * inv[None, :]
    cos = jnp.cos(ang)[None, :, None, :]
    sin = jnp.sin(ang)[None, :, None, :]
    x1, x2 = x[..., :half], x[..., half:]
    return jnp.concatenate([x1 * cos - x2 * sin, x2 * cos + x1 * sin], axis=-1)


def _router_body(xf_ref, rw_ref, w_ref):
    rl = _mm(xf_ref[...], rw_ref[...])
    mm = jnp.max(rl, axis=-1, keepdims=True)
    pe = jnp.exp(rl - mm)
    pr = pe / jnp.sum(pe, axis=-1, keepdims=True)
    ii = lax.broadcasted_iota(jnp.int32, (BT, E), 1)
    m1 = jnp.max(pr, axis=-1, keepdims=True)
    i1 = jnp.min(jnp.where(pr >= m1, ii, E), axis=-1, keepdims=True)
    first1 = ii == i1
    pm = jnp.where(first1, -1.0, pr)
    m2 = jnp.max(pm, axis=-1, keepdims=True)
    i2 = jnp.min(jnp.where(pm >= m2, ii, E), axis=-1, keepdims=True)
    first2 = ii == i2
    tot = m1 + m2
    w_ref[...] = jnp.where(first1, m1 / tot, 0.0) + jnp.where(first2, m2 / tot, 0.0)


def _router(xf, rw):
    return pl.pallas_call(
        _router_body,
        grid=(NT,),
        in_specs=[
            pl.BlockSpec((BT, D), lambda t: (t, 0)),
            pl.BlockSpec((D, E), lambda t: (0, 0)),
        ],
        out_specs=pl.BlockSpec((BT, E), lambda t: (t, 0)),
        out_shape=jax.ShapeDtypeStruct((S, E), f32),
    )(xf, rw)


def _moe_body(xf_ref, wg_ref, wu_ref, wd_ref, w3_ref, y_ref):
    e = pl.program_id(1)
    x = xf_ref[...]
    g = _mm(x, wg_ref[0])
    u = _mm(x, wu_ref[0])
    a = (g / (1.0 + jnp.exp(-g))) * u
    eo = _mm(a, wd_ref[0])
    contrib = _bf(w3_ref[0]).astype(f32) * _bf(eo).astype(f32)

    @pl.when(e == 0)
    def _():
        y_ref[...] = contrib

    @pl.when(e > 0)
    def _():
        y_ref[...] += contrib


def _moe(xf, Wg, Wu, Wd, w3):
    return pl.pallas_call(
        _moe_body,
        grid=(NTM, E),
        in_specs=[
            pl.BlockSpec((BTM, D), lambda t, e: (t, 0)),
            pl.BlockSpec((1, D, F), lambda t, e: (e, 0, 0)),
            pl.BlockSpec((1, D, F), lambda t, e: (e, 0, 0)),
            pl.BlockSpec((1, F, D), lambda t, e: (e, 0, 0)),
            pl.BlockSpec((1, BTM, 1), lambda t, e: (e, t, 0)),
        ],
        out_specs=pl.BlockSpec((BTM, D), lambda t, e: (t, 0)),
        out_shape=jax.ShapeDtypeStruct((S, D), f32),
    )(xf, Wg, Wu, Wd, w3)


def _shared_body(xf_ref, hn_ref, y_ref, wsg_ref, wsu_ref, wsd_ref, sg_ref, out_ref):
    x = xf_ref[...]
    g = _mm(x, wsg_ref[...])
    u = _mm(x, wsu_ref[...])
    a = (g / (1.0 + jnp.exp(-g))) * u
    sh = _mm(a, wsd_ref[...])
    gate_l = jnp.sum(_bf(x).astype(f32) * _bf(sg_ref[...]).astype(f32),
                     axis=-1, keepdims=True)
    gate = 1.0 / (1.0 + jnp.exp(-gate_l))
    out_ref[...] = hn_ref[...] + y_ref[...] + gate * sh


def _shared(xf, hn, y, Wsg, Wsu, Wsd, sg):
    return pl.pallas_call(
        _shared_body,
        grid=(NT,),
        in_specs=[
            pl.BlockSpec((BT, D), lambda t: (t, 0)),
            pl.BlockSpec((BT, D), lambda t: (t, 0)),
            pl.BlockSpec((BT, D), lambda t: (t, 0)),
            pl.BlockSpec((D, FS), lambda t: (0, 0)),
            pl.BlockSpec((D, FS), lambda t: (0, 0)),
            pl.BlockSpec((FS, D), lambda t: (0, 0)),
            pl.BlockSpec((1, D), lambda t: (0, 0)),
        ],
        out_specs=pl.BlockSpec((BT, D), lambda t: (t, 0)),
        out_shape=jax.ShapeDtypeStruct((S, D), f32),
    )(xf, hn, y, Wsg, Wsu, Wsd, sg)


def _final_body(h_ref, w_ref, out_ref):
    out_ref[...] = _mm(h_ref[...], w_ref[...])


def _final(h, lm_head):
    return pl.pallas_call(
        _final_body,
        grid=(NV,),
        in_specs=[
            pl.BlockSpec((S, D), lambda v: (0, 0)),
            pl.BlockSpec((D, BV), lambda v: (0, v)),
        ],
        out_specs=pl.BlockSpec((S, BV), lambda v: (0, v)),
        out_shape=jax.ShapeDtypeStruct((S, V), f32),
    )(h, lm_head)


def kernel(input_ids, embed, attn_norm_g, Wq, Wk, Wv, Wo, ffn_norm_g, router_w,
           We_gate, We_up, We_down, Ws_gate_w, Ws_up_w, Ws_down_w, shared_gate_w,
           final_norm_g, lm_head):
    h = jnp.take(embed, input_ids, axis=0)
    pos = jnp.arange(S, dtype=jnp.float32)
    causal = jnp.tril(jnp.ones((S, S), dtype=bool))[None, None]

    for l in range(L):
        x = _rms(h, attn_norm_g[l])
        q = _rope((x @ Wq[l]).reshape(B, S, H, HD), pos)
        k = _rope((x @ Wk[l]).reshape(B, S, KVH, HD), pos)
        v = (x @ Wv[l]).reshape(B, S, KVH, HD)
        k = jnp.repeat(k, H // KVH, axis=2)
        v = jnp.repeat(v, H // KVH, axis=2)
        sc = jnp.einsum('bqhd,bkhd->bhqk', q, k) / jnp.sqrt(jnp.float32(HD))
        sc = jnp.where(causal, sc, jnp.float32(-1e9))
        att = jax.nn.softmax(sc, axis=-1)
        o = jnp.einsum('bhqk,bkhd->bqhd', att, v).reshape(B, S, H * HD) @ Wo[l]
        h = h + o
        xf = _rms(h, ffn_norm_g[l]).reshape(B * S, D)
        if l < L - 1:
            rl = xf @ router_w[l]
            pr = jax.nn.softmax(rl, axis=-1)
            tv, ti = jax.lax.top_k(pr, 2)
            tv = tv / jnp.sum(tv, axis=-1, keepdims=True)
            w = jnp.zeros((B * S, E), jnp.float32).at[
                jnp.arange(B * S)[:, None], ti].add(tv)
            g = jnp.einsum('td,edf->tef', xf, We_gate[l])
            u = jnp.einsum('td,edf->tef', xf, We_up[l])
            eo = jnp.einsum('tef,efd->ted', jax.nn.silu(g) * u, We_down[l])
            y = jnp.einsum('te,ted->td', w, eo)
            sh = (jax.nn.silu(xf @ Ws_gate_w[l]) * (xf @ Ws_up_w[l])) @ Ws_down_w[l]
            y = y + jax.nn.sigmoid(xf @ shared_gate_w[l]) * sh
            h = h + y.reshape(B, S, D)
        else:
            w = _router(xf, router_w[l])
            w3 = w.T.reshape(E, S, 1)
            y = _moe(xf, We_gate[l], We_up[l], We_down[l], w3)
            h = _shared(xf, h.reshape(S, D), y, Ws_gate_w[l], Ws_up_w[l],
                        Ws_down_w[l], shared_gate_w[l].reshape(1, D))

    h = _rms(h, final_norm_g).reshape(S, D)
    out = _final(h, lm_head)
    return out.reshape(B, S, V)


# speedup vs baseline: 1.0147x; 1.0147x over previous
import jax
import jax.numpy as jnp
from jax import lax
from jax.experimental import pallas as pl
V = 32000; D = 1024; L = 2; H = 16; KVH = 4; HD = 64
E = 8; F = 512; FS = 512; B = 1; S = 2048
THETA = 1000000.0; EPS = 1e-6
BT = 256
NT = S // BT
BV = 1280
NV = V // BV
BTM = 1024
NTM = S // BTM

f32 = jnp.float32
bf16 = jnp.bfloat16


def _bf(a):
    return a.astype(bf16)


def _mm(a, b):
    return jnp.dot(_bf(a), _bf(b), preferred_element_type=f32)


def _rms(x, g):
    return x * lax.rsqrt(jnp.mean(x * x, axis=-1, keepdims=True) + EPS) * g


def _rope(x, pos):
    half = HD // 2
    inv = 1.0 / (THETA ** (jnp.arange(0, half, dtype=jnp.float32) / half))
    ang = pos[:, None] * inv[None, :]
    cos = jnp.cos(ang)[None, :, None, :]
    sin = jnp.sin(ang)[None, :, None, :]
    x1, x2 = x[..., :half], x[..., half:]
    return jnp.concatenate([x1 * cos - x2 * sin, x2 * cos + x1 * sin], axis=-1)


def _router_body(xf_ref, rw_ref, w_ref):
    rl = _mm(xf_ref[...], rw_ref[...])
    mm = jnp.max(rl, axis=-1, keepdims=True)
    pe = jnp.exp(rl - mm)
    pr = pe / jnp.sum(pe, axis=-1, keepdims=True)
    ii = lax.broadcasted_iota(jnp.int32, (BT, E), 1)
    m1 = jnp.max(pr, axis=-1, keepdims=True)
    i1 = jnp.min(jnp.where(pr >= m1, ii, E), axis=-1, keepdims=True)
    first1 = ii == i1
    pm = jnp.where(first1, -1.0, pr)
    m2 = jnp.max(pm, axis=-1, keepdims=True)
    i2 = jnp.min(jnp.where(pm >= m2, ii, E), axis=-1, keepdims=True)
    first2 = ii == i2
    tot = m1 + m2
    w_ref[...] = jnp.where(first1, m1 / tot, 0.0) + jnp.where(first2, m2 / tot, 0.0)


def _router(xf, rw):
    return pl.pallas_call(
        _router_body,
        grid=(NT,),
        in_specs=[
            pl.BlockSpec((BT, D), lambda t: (t, 0)),
            pl.BlockSpec((D, E), lambda t: (0, 0)),
        ],
        out_specs=pl.BlockSpec((BT, E), lambda t: (t, 0)),
        out_shape=jax.ShapeDtypeStruct((S, E), f32),
    )(xf, rw)


def _moe_body(xf_ref, wg_ref, wu_ref, wd_ref, w3_ref, y_ref):
    e = pl.program_id(1)
    x = xf_ref[...]
    g = _mm(x, wg_ref[0])
    u = _mm(x, wu_ref[0])
    a = (g / (1.0 + jnp.exp(-g))) * u
    eo = _mm(a, wd_ref[0])
    contrib = _bf(w3_ref[0]).astype(f32) * _bf(eo).astype(f32)

    @pl.when(e == 0)
    def _():
        y_ref[...] = contrib

    @pl.when(e > 0)
    def _():
        y_ref[...] += contrib


def _moe(xf, Wg, Wu, Wd, w3):
    return pl.pallas_call(
        _moe_body,
        grid=(NTM, E),
        in_specs=[
            pl.BlockSpec((BTM, D), lambda t, e: (t, 0)),
            pl.BlockSpec((1, D, F), lambda t, e: (e, 0, 0)),
            pl.BlockSpec((1, D, F), lambda t, e: (e, 0, 0)),
            pl.BlockSpec((1, F, D), lambda t, e: (e, 0, 0)),
            pl.BlockSpec((1, BTM, 1), lambda t, e: (e, t, 0)),
        ],
        out_specs=pl.BlockSpec((BTM, D), lambda t, e: (t, 0)),
        out_shape=jax.ShapeDtypeStruct((S, D), f32),
    )(xf, Wg, Wu, Wd, w3)


def _shared_body(xf_ref, hn_ref, y_ref, wsg_ref, wsu_ref, wsd_ref, sg_ref, out_ref):
    x = xf_ref[...]
    g = _mm(x, wsg_ref[...])
    u = _mm(x, wsu_ref[...])
    a = (g / (1.0 + jnp.exp(-g))) * u
    sh = _mm(a, wsd_ref[...])
    gate_l = jnp.sum(_bf(x).astype(f32) * _bf(sg_ref[...]).astype(f32),
                     axis=-1, keepdims=True)
    gate = 1.0 / (1.0 + jnp.exp(-gate_l))
    out_ref[...] = hn_ref[...] + y_ref[...] + gate * sh


def _shared(xf, hn, y, Wsg, Wsu, Wsd, sg):
    return pl.pallas_call(
        _shared_body,
        grid=(NT,),
        in_specs=[
            pl.BlockSpec((BT, D), lambda t: (t, 0)),
            pl.BlockSpec((BT, D), lambda t: (t, 0)),
            pl.BlockSpec((BT, D), lambda t: (t, 0)),
            pl.BlockSpec((D, FS), lambda t: (0, 0)),
            pl.BlockSpec((D, FS), lambda t: (0, 0)),
            pl.BlockSpec((FS, D), lambda t: (0, 0)),
            pl.BlockSpec((1, D), lambda t: (0, 0)),
        ],
        out_specs=pl.BlockSpec((BT, D), lambda t: (t, 0)),
        out_shape=jax.ShapeDtypeStruct((S, D), f32),
    )(xf, hn, y, Wsg, Wsu, Wsd, sg)


def _final_body(h_ref, w_ref, out_ref):
    out_ref[...] = _mm(h_ref[...], w_ref[...])


def _final(h, lm_head):
    return pl.pallas_call(
        _final_body,
        grid=(NV,),
        in_specs=[
            pl.BlockSpec((S, D), lambda v: (0, 0)),
            pl.BlockSpec((D, BV), lambda v: (0, v)),
        ],
        out_specs=pl.BlockSpec((S, BV), lambda v: (0, v)),
        out_shape=jax.ShapeDtypeStruct((S, V), f32),
    )(h, lm_head)


def kernel(input_ids, embed, attn_norm_g, Wq, Wk, Wv, Wo, ffn_norm_g, router_w,
           We_gate, We_up, We_down, Ws_gate_w, Ws_up_w, Ws_down_w, shared_gate_w,
           final_norm_g, lm_head):
    h = jnp.take(embed, input_ids, axis=0)
    pos = jnp.arange(S, dtype=jnp.float32)
    causal = jnp.tril(jnp.ones((S, S), dtype=bool))[None, None]

    for l in range(L):
        x = _rms(h, attn_norm_g[l])
        q = _rope((x @ Wq[l]).reshape(B, S, H, HD), pos)
        k = _rope((x @ Wk[l]).reshape(B, S, KVH, HD), pos)
        v = (x @ Wv[l]).reshape(B, S, KVH, HD)
        k = jnp.repeat(k, H // KVH, axis=2)
        v = jnp.repeat(v, H // KVH, axis=2)
        sc = jnp.einsum('bqhd,bkhd->bhqk', q, k) / jnp.sqrt(jnp.float32(HD))
        sc = jnp.where(causal, sc, jnp.float32(-1e9))
        att = jax.nn.softmax(sc, axis=-1)
        o = jnp.einsum('bhqk,bkhd->bqhd', att, v).reshape(B, S, H * HD) @ Wo[l]
        h = h + o
        xf = _rms(h, ffn_norm_g[l]).reshape(B * S, D)
        if l < L - 1:
            rl = xf @ router_w[l]
            pr = jax.nn.softmax(rl, axis=-1)
            tv, ti = jax.lax.top_k(pr, 2)
            tv = tv / jnp.sum(tv, axis=-1, keepdims=True)
            w = jnp.zeros((B * S, E), jnp.float32).at[
                jnp.arange(B * S)[:, None], ti].add(tv)
            g = jnp.einsum('td,edf->tef', xf, We_gate[l])
            u = jnp.einsum('td,edf->tef', xf, We_up[l])
            eo = jnp.einsum('tef,efd->ted', jax.nn.silu(g) * u, We_down[l])
            y = jnp.einsum('te,ted->td', w, eo)
            sh = (jax.nn.silu(xf @ Ws_gate_w[l]) * (xf @ Ws_up_w[l])) @ Ws_down_w[l]
            y = y + jax.nn.sigmoid(xf @ shared_gate_w[l]) * sh
            h = h + y.reshape(B, S, D)
        else:
            w = _router(xf, router_w[l])
            w3 = w.T.reshape(E, S, 1)
            y = _moe(xf, We_gate[l], We_up[l], We_down[l], w3)
            h = _shared(xf, h.reshape(S, D), y, Ws_gate_w[l], Ws_up_w[l],
                        Ws_down_w[l], shared_gate_w[l].reshape(1, D))

    h = _rms(h, final_norm_g).reshape(S, D)
    out = _final(h, lm_head)
    return out.reshape(B, S, V)
